# Initial kernel scaffold; baseline (speedup 1.0000x reference)
import jax
import jax.numpy as jnp
from jax.experimental import pallas as pl


def kernel(x, edge_index):
    raise NotImplementedError("write your pallas kernel here")


# speedup vs baseline: 2.5810x; 2.5810x over previous
import functools

import jax
import jax.numpy as jnp
from jax import lax
from jax.experimental import pallas as pl
from jax.experimental.pallas import tpu as pltpu
from jax.experimental.pallas import tpu_sc as plsc

N_NODES = 10000
N_EDGES = 320000
D_FEAT = 128

NC = 2
NS = 16
NW = NC * NS
L = 16

E_PER_W = N_EDGES // NW
CHUNK = 80
N_CHUNKS = E_PER_W // CHUNK
GROUPS = CHUNK // L
DK = D_FEAT // L


def _edge_scores_body(x_hbm, src_hbm, dst_hbm, out_hbm,
                      idx_u, idx_v, hu, hv, outb, sem_u, sem_v):
    wid = lax.axis_index("s") * NC + lax.axis_index("c")
    w_base = wid * E_PER_W

    lane = lax.broadcasted_iota(jnp.int32, (L,), 0)

    def chunk_body(ci, carry):
        base = w_base + ci * CHUNK
        pltpu.sync_copy(src_hbm.at[pl.ds(base, CHUNK)], idx_u)
        pltpu.sync_copy(dst_hbm.at[pl.ds(base, CHUNK)], idx_v)
        cp_u = pltpu.async_copy(x_hbm.at[idx_u], hu, sem_u)
        cp_v = pltpu.async_copy(x_hbm.at[idx_v], hv, sem_v)
        cp_u.wait()
        cp_v.wait()

        def group_body(g, c2):
            e0 = g * L
            score = jnp.zeros((L,), jnp.float32)
            for e in range(L):
                ei = e0 + e
                acc = hu[ei, pl.ds(0, L)] * hv[ei, pl.ds(0, L)]
                for k in range(1, DK):
                    acc = acc + hu[ei, pl.ds(k * L, L)] * hv[ei, pl.ds(k * L, L)]
                score = jnp.where(lane == e, jnp.sum(acc), score)
            outb[pl.ds(e0, L)] = 1.0 / (1.0 + jnp.exp(-score))
            return c2

        lax.fori_loop(0, GROUPS, group_body, 0, unroll=False)
        pltpu.sync_copy(outb, out_hbm.at[pl.ds(base, CHUNK)])
        return carry

    lax.fori_loop(0, N_CHUNKS, chunk_body, 0, unroll=False)


@jax.jit
def _edge_scores(x, src, dst):
    mesh = plsc.VectorSubcoreMesh(core_axis_name="c", subcore_axis_name="s")
    fn = pl.kernel(
        _edge_scores_body,
        mesh=mesh,
        compiler_params=pltpu.CompilerParams(needs_layout_passes=False),
        out_type=jax.ShapeDtypeStruct((N_EDGES,), jnp.float32),
        scratch_types=[
            pltpu.VMEM((CHUNK,), jnp.int32),
            pltpu.VMEM((CHUNK,), jnp.int32),
            pltpu.VMEM((CHUNK, D_FEAT), jnp.float32),
            pltpu.VMEM((CHUNK, D_FEAT), jnp.float32),
            pltpu.VMEM((CHUNK,), jnp.float32),
            pltpu.SemaphoreType.DMA,
            pltpu.SemaphoreType.DMA,
        ],
    )
    return fn(x, src, dst)


def kernel(x, edge_index):
    src = edge_index[0]
    dst = edge_index[1]
    return _edge_scores(x, src, dst)


# speedup vs baseline: 3.3379x; 1.2933x over previous
import jax
import jax.numpy as jnp
from jax import lax
from jax.experimental import pallas as pl
from jax.experimental.pallas import tpu as pltpu
from jax.experimental.pallas import tpu_sc as plsc

N_NODES = 10000
N_EDGES = 320000
D_FEAT = 128

NC = 2
NS = 16
NW = NC * NS
L = 16

E_PER_W = N_EDGES // NW
CHUNK = 80
N_CHUNKS = E_PER_W // CHUNK
GROUPS = CHUNK // L
DK = D_FEAT // L


def _edge_scores_body(x_hbm, src_hbm, dst_hbm, out_hbm,
                      idx_u0, idx_v0, idx_u1, idx_v1,
                      hu0, hv0, hu1, hv1, outb0, outb1,
                      sem_u0, sem_v0, sem_u1, sem_v1, sem_o0, sem_o1):
    idx_u = (idx_u0, idx_u1)
    idx_v = (idx_v0, idx_v1)
    hu = (hu0, hu1)
    hv = (hv0, hv1)
    outb = (outb0, outb1)
    sem_u = (sem_u0, sem_u1)
    sem_v = (sem_v0, sem_v1)
    sem_o = (sem_o0, sem_o1)

    wid = lax.axis_index("s") * NC + lax.axis_index("c")
    w_base = wid * E_PER_W
    lane = lax.broadcasted_iota(jnp.int32, (L,), 0)

    def issue(ci, b):
        base = w_base + ci * CHUNK
        pltpu.sync_copy(src_hbm.at[pl.ds(base, CHUNK)], idx_u[b])
        pltpu.sync_copy(dst_hbm.at[pl.ds(base, CHUNK)], idx_v[b])
        pltpu.async_copy(x_hbm.at[idx_u[b]], hu[b], sem_u[b])
        pltpu.async_copy(x_hbm.at[idx_v[b]], hv[b], sem_v[b])

    def wait_gathers(b):
        pltpu.make_async_copy(x_hbm.at[idx_u[b]], hu[b], sem_u[b]).wait()
        pltpu.make_async_copy(x_hbm.at[idx_v[b]], hv[b], sem_v[b]).wait()

    def start_out(ci, b):
        base = w_base + ci * CHUNK
        pltpu.async_copy(outb[b], out_hbm.at[pl.ds(base, CHUNK)], sem_o[b])

    def wait_out(ci, b):
        base = w_base + ci * CHUNK
        pltpu.make_async_copy(
            outb[b], out_hbm.at[pl.ds(base, CHUNK)], sem_o[b]).wait()

    def compute(b):
        hub, hvb, outbb = hu[b], hv[b], outb[b]

        def group_body(g, c2):
            e0 = g * L
            score = jnp.zeros((L,), jnp.float32)
            for e in range(L):
                ei = e0 + e
                acc = hub[ei, pl.ds(0, L)] * hvb[ei, pl.ds(0, L)]
                for k in range(1, DK):
                    acc = acc + hub[ei, pl.ds(k * L, L)] * hvb[ei, pl.ds(k * L, L)]
                score = jnp.where(lane == e, jnp.sum(acc), score)
            outbb[pl.ds(e0, L)] = 1.0 / (1.0 + jnp.exp(-score))
            return c2

        lax.fori_loop(0, GROUPS, group_body, 0, unroll=False)

    issue(0, 0)

    def pair_body(i, carry):
        ci = 2 * i
        issue(ci + 1, 1)
        wait_gathers(0)

        @pl.when(ci >= 2)
        def _():
            wait_out(ci - 2, 0)

        compute(0)
        start_out(ci, 0)

        issue(ci + 2, 0)
        wait_gathers(1)

        @pl.when(ci >= 1)
        def _():
            wait_out(ci - 1, 1)

        compute(1)
        start_out(ci + 1, 1)
        return carry

    lax.fori_loop(0, (N_CHUNKS - 1) // 2, pair_body, 0, unroll=False)

    last = N_CHUNKS - 1
    wait_gathers(0)
    wait_out(last - 2, 0)
    compute(0)
    start_out(last, 0)
    wait_out(last - 1, 1)
    wait_out(last, 0)


@jax.jit
def _edge_scores(x, src, dst):
    mesh = plsc.VectorSubcoreMesh(core_axis_name="c", subcore_axis_name="s")
    fn = pl.kernel(
        _edge_scores_body,
        mesh=mesh,
        compiler_params=pltpu.CompilerParams(needs_layout_passes=False),
        out_type=jax.ShapeDtypeStruct((N_EDGES,), jnp.float32),
        scratch_types=[
            pltpu.VMEM((CHUNK,), jnp.int32),
            pltpu.VMEM((CHUNK,), jnp.int32),
            pltpu.VMEM((CHUNK,), jnp.int32),
            pltpu.VMEM((CHUNK,), jnp.int32),
            pltpu.VMEM((CHUNK, D_FEAT), jnp.float32),
            pltpu.VMEM((CHUNK, D_FEAT), jnp.float32),
            pltpu.VMEM((CHUNK, D_FEAT), jnp.float32),
            pltpu.VMEM((CHUNK, D_FEAT), jnp.float32),
            pltpu.VMEM((CHUNK,), jnp.float32),
            pltpu.VMEM((CHUNK,), jnp.float32),
            pltpu.SemaphoreType.DMA,
            pltpu.SemaphoreType.DMA,
            pltpu.SemaphoreType.DMA,
            pltpu.SemaphoreType.DMA,
            pltpu.SemaphoreType.DMA,
            pltpu.SemaphoreType.DMA,
        ],
    )
    return fn(x, src, dst)


def kernel(x, edge_index):
    src = edge_index[0]
    dst = edge_index[1]
    return _edge_scores(x, src, dst)
